# baseline (speedup 1.0000x reference)
import jax
import jax.numpy as jnp
from jax import lax
from jax.experimental import pallas as pl
from jax.experimental.pallas import tpu as pltpu
from jax.experimental.pallas import tpu_sc as plsc

EDIM = 64
HD = EDIM // 2
NC, NS = 2, 16
NW = NC * NS
SEG = 64
VB = 8192
QV = VB // 4
MML = 10


def _rne16(u):
    return u + jnp.uint32(0x7FFF) + ((u >> 16) & jnp.uint32(1))


def _transpose_body(x_ref, o_ref):
    x = x_ref[...]
    ulo = lax.bitcast_convert_type(x[:HD, :], jnp.uint32)
    uhi = lax.bitcast_convert_type(x[HD:, :], jnp.uint32)
    packed_u = (_rne16(ulo) >> 16) | (_rne16(uhi) & jnp.uint32(0xFFFF0000))
    packed = lax.bitcast_convert_type(packed_u, jnp.float32)
    for t in range(4):
        o_ref[:, 32 * t:32 * (t + 1)] = packed[:, t * QV:(t + 1) * QV].T


def _tc_table_pack(table_t):
    v = table_t.shape[1]
    nb = pl.cdiv(v, VB)
    return pl.pallas_call(
        _transpose_body,
        grid=(nb,),
        in_specs=[pl.BlockSpec((EDIM, VB), lambda i: (0, i))],
        out_specs=pl.BlockSpec((QV, 128), lambda i: (i, 0)),
        out_shape=jax.ShapeDtypeStruct((nb * QV, 128), jnp.float32),
    )(table_t)


def _gather_body(idx_hbm, table_hbm, out_hbm,
                 ichunk_a, ichunk_b, rows0, rows1, sidx_a, sidx_b,
                 sg, sow0, sow1):
    n4 = out_hbm.shape[0]
    ch_tot = n4 // SEG
    ch_w = ch_tot // NW
    chunks_per_l = idx_hbm.shape[1] // SEG
    wid = lax.axis_index("s") * NC + lax.axis_index("c")
    cbase = wid * ch_w

    def idx_src(c):
        g = cbase + c
        l = g // chunks_per_l
        j = g % chunks_per_l
        return idx_hbm.at[pl.ds(4 * l, 4), pl.ds(SEG * j, SEG)]

    def fire_idx(c, ibuf, sem):
        @pl.when(c < ch_w)
        def _():
            pltpu.async_copy(idx_src(c), ibuf, sem)

    fire_idx(0, ichunk_a, sidx_a)

    def step(p, carry):
        for k, (ibuf, sidx, rows, sow) in enumerate((
                (ichunk_a, sidx_a, rows0, sow0),
                (ichunk_b, sidx_b, rows1, sow1))):
            c = 2 * p + k

            @pl.when(c < ch_w)
            def _():
                g = cbase + c
                fire_idx(c + 1, ichunk_b if k == 0 else ichunk_a,
                         sidx_b if k == 0 else sidx_a)
                pltpu.make_async_copy(idx_src(c), ibuf, sidx).wait()

                @pl.when(c >= 2)
                def _():
                    for h in range(4):
                        pltpu.make_async_copy(
                            rows[h], out_hbm.at[pl.ds(g * SEG, SEG), h],
                            sow).wait()

                for h in range(4):
                    pltpu.async_copy(table_hbm.at[ibuf.at[h]], rows[h], sg)
                for h in range(4):
                    pltpu.make_async_copy(
                        table_hbm.at[ibuf.at[h]], rows[h], sg).wait()
                for h in range(4):
                    pltpu.async_copy(
                        rows[h], out_hbm.at[pl.ds(g * SEG, SEG), h], sow)
        return carry

    lax.fori_loop(0, (ch_w + 1) // 2, step, 0)
    for rows, sow in ((rows0, sow0), (rows1, sow1)):
        for h in range(4):
            pltpu.make_async_copy(
                rows[h], out_hbm.at[pl.ds(0, SEG), h], sow).wait()


def _sc_gather(idx_4d, table_flat):
    n = idx_4d.shape[0] * idx_4d.shape[1]
    mesh = plsc.VectorSubcoreMesh(core_axis_name="c", subcore_axis_name="s")
    return pl.kernel(
        _gather_body,
        out_type=jax.ShapeDtypeStruct((n // 4, 4, HD), jnp.float32),
        mesh=mesh,
        scratch_types=[
            pltpu.VMEM((4, SEG), jnp.int32),
            pltpu.VMEM((4, SEG), jnp.int32),
            [pltpu.VMEM((SEG, HD), jnp.float32)] * 4,
            [pltpu.VMEM((SEG, HD), jnp.float32)] * 4,
            pltpu.SemaphoreType.DMA,
            pltpu.SemaphoreType.DMA,
            pltpu.SemaphoreType.DMA,
            pltpu.SemaphoreType.DMA,
            pltpu.SemaphoreType.DMA,
        ],
        compiler_params=pltpu.CompilerParams(use_tc_tiling_on_sc=False),
    )(idx_4d, table_flat)


def _mm_body(x_ref, w_ref, o_ref):
    qb = x_ref.shape[1]
    w = w_ref[...]
    we = w[:, :HD]
    wo = w[:, HD:]
    dn = (((1,), (1,)), ((), ()))
    for s in range(MML):
        u = lax.bitcast_convert_type(x_ref[s], jnp.uint32)
        xe = lax.bitcast_convert_type(u << 16, jnp.float32)
        xo = lax.bitcast_convert_type(u & jnp.uint32(0xFFFF0000), jnp.float32)
        for t in range(4):
            cs = slice(32 * t, 32 * (t + 1))
            o_ref[s, :, t * qb:(t + 1) * qb] = (
                jax.lax.dot_general(we, xe[:, cs], dn,
                                    preferred_element_type=jnp.float32)
                + jax.lax.dot_general(wo, xo[:, cs], dn,
                                      preferred_element_type=jnp.float32))


def _tc_project_t(emb, w, l, b):
    x128 = emb.reshape(l, b // 4, 4 * HD)
    return pl.pallas_call(
        _mm_body,
        grid=(l // MML,),
        in_specs=[
            pl.BlockSpec((MML, b // 4, 4 * HD), lambda i: (i, 0, 0)),
            pl.BlockSpec((EDIM, EDIM), lambda i: (0, 0)),
        ],
        out_specs=pl.BlockSpec((MML, EDIM, b), lambda i: (i, 0, 0)),
        out_shape=jax.ShapeDtypeStruct((l, EDIM, b), jnp.float32),
    )(x128, w)


def kernel(prem, hypo, table, W):
    B, L = prem.shape
    pairs = _tc_table_pack(table.T)
    table_flat = pairs.reshape(pairs.shape[0] * 4, HD)
    outs = []
    for ind in (prem, hypo):
        idx = ind.T.reshape(4 * L, B // 4)
        ridx = (idx // VB) * VB + 4 * (idx % QV) + (idx % VB) // QV
        emb = _sc_gather(ridx, table_flat)
        out_t = _tc_project_t(emb, W, L, B)
        outs.append(out_t.transpose(2, 0, 1))
    return (outs[0], outs[1])


# speedup vs baseline: 1.0027x; 1.0027x over previous
import jax
import jax.numpy as jnp
from jax import lax
from jax.experimental import pallas as pl
from jax.experimental.pallas import tpu as pltpu
from jax.experimental.pallas import tpu_sc as plsc

EDIM = 64
HD = EDIM // 2
NC, NS = 2, 16
NW = NC * NS
SEG = 64
VB = 8192
QV = VB // 4
MML = 5


def _rne16(u):
    return u + jnp.uint32(0x7FFF) + ((u >> 16) & jnp.uint32(1))


def _transpose_body(x_ref, o_ref):
    x = x_ref[...]
    ulo = lax.bitcast_convert_type(x[:HD, :], jnp.uint32)
    uhi = lax.bitcast_convert_type(x[HD:, :], jnp.uint32)
    packed_u = (_rne16(ulo) >> 16) | (_rne16(uhi) & jnp.uint32(0xFFFF0000))
    packed = lax.bitcast_convert_type(packed_u, jnp.float32)
    for t in range(4):
        o_ref[:, 32 * t:32 * (t + 1)] = packed[:, t * QV:(t + 1) * QV].T


def _tc_table_pack(table_t):
    v = table_t.shape[1]
    nb = pl.cdiv(v, VB)
    return pl.pallas_call(
        _transpose_body,
        grid=(nb,),
        in_specs=[pl.BlockSpec((EDIM, VB), lambda i: (0, i))],
        out_specs=pl.BlockSpec((QV, 128), lambda i: (i, 0)),
        out_shape=jax.ShapeDtypeStruct((nb * QV, 128), jnp.float32),
    )(table_t)


def _gather_body(idx_hbm, table_hbm, out_hbm,
                 ichunk_a, ichunk_b, rows0, rows1, sidx_a, sidx_b,
                 sg, sow0, sow1):
    n4 = out_hbm.shape[0]
    ch_tot = n4 // SEG
    ch_w = ch_tot // NW
    chunks_per_l = idx_hbm.shape[1] // SEG
    wid = lax.axis_index("s") * NC + lax.axis_index("c")
    cbase = wid * ch_w

    def idx_src(c):
        g = cbase + c
        l = g // chunks_per_l
        j = g % chunks_per_l
        return idx_hbm.at[pl.ds(4 * l, 4), pl.ds(SEG * j, SEG)]

    def fire_idx(c, ibuf, sem):
        @pl.when(c < ch_w)
        def _():
            pltpu.async_copy(idx_src(c), ibuf, sem)

    fire_idx(0, ichunk_a, sidx_a)

    def step(p, carry):
        for k, (ibuf, sidx, rows, sow) in enumerate((
                (ichunk_a, sidx_a, rows0, sow0),
                (ichunk_b, sidx_b, rows1, sow1))):
            c = 2 * p + k

            @pl.when(c < ch_w)
            def _():
                g = cbase + c
                fire_idx(c + 1, ichunk_b if k == 0 else ichunk_a,
                         sidx_b if k == 0 else sidx_a)
                pltpu.make_async_copy(idx_src(c), ibuf, sidx).wait()

                @pl.when(c >= 2)
                def _():
                    for h in range(4):
                        pltpu.make_async_copy(
                            rows[h], out_hbm.at[pl.ds(g * SEG, SEG), h],
                            sow).wait()

                for h in range(4):
                    pltpu.async_copy(table_hbm.at[ibuf.at[h]], rows[h], sg)
                for h in range(4):
                    pltpu.make_async_copy(
                        table_hbm.at[ibuf.at[h]], rows[h], sg).wait()
                for h in range(4):
                    pltpu.async_copy(
                        rows[h], out_hbm.at[pl.ds(g * SEG, SEG), h], sow)
        return carry

    lax.fori_loop(0, (ch_w + 1) // 2, step, 0)
    for rows, sow in ((rows0, sow0), (rows1, sow1)):
        for h in range(4):
            pltpu.make_async_copy(
                rows[h], out_hbm.at[pl.ds(0, SEG), h], sow).wait()


def _sc_gather(idx_4d, table_flat):
    n = idx_4d.shape[0] * idx_4d.shape[1]
    mesh = plsc.VectorSubcoreMesh(core_axis_name="c", subcore_axis_name="s")
    return pl.kernel(
        _gather_body,
        out_type=jax.ShapeDtypeStruct((n // 4, 4, HD), jnp.float32),
        mesh=mesh,
        scratch_types=[
            pltpu.VMEM((4, SEG), jnp.int32),
            pltpu.VMEM((4, SEG), jnp.int32),
            [pltpu.VMEM((SEG, HD), jnp.float32)] * 4,
            [pltpu.VMEM((SEG, HD), jnp.float32)] * 4,
            pltpu.SemaphoreType.DMA,
            pltpu.SemaphoreType.DMA,
            pltpu.SemaphoreType.DMA,
            pltpu.SemaphoreType.DMA,
            pltpu.SemaphoreType.DMA,
        ],
        compiler_params=pltpu.CompilerParams(use_tc_tiling_on_sc=False),
    )(idx_4d, table_flat)


def _mm_body(x_ref, w_ref, o_ref):
    qb = x_ref.shape[1]
    w = w_ref[...]
    we = w[:, :HD]
    wo = w[:, HD:]
    dn = (((1,), (1,)), ((), ()))
    for s in range(MML):
        u = lax.bitcast_convert_type(x_ref[s], jnp.uint32)
        xe = lax.bitcast_convert_type(u << 16, jnp.float32)
        xo = lax.bitcast_convert_type(u & jnp.uint32(0xFFFF0000), jnp.float32)
        for t in range(4):
            cs = slice(32 * t, 32 * (t + 1))
            o_ref[s, :, t * qb:(t + 1) * qb] = (
                jax.lax.dot_general(we, xe[:, cs], dn,
                                    preferred_element_type=jnp.float32)
                + jax.lax.dot_general(wo, xo[:, cs], dn,
                                      preferred_element_type=jnp.float32))


def _tc_project_t(emb, w, l, b):
    x128 = emb.reshape(l, b // 4, 4 * HD)
    return pl.pallas_call(
        _mm_body,
        grid=(l // MML,),
        in_specs=[
            pl.BlockSpec((MML, b // 4, 4 * HD), lambda i: (i, 0, 0)),
            pl.BlockSpec((EDIM, EDIM), lambda i: (0, 0)),
        ],
        out_specs=pl.BlockSpec((MML, EDIM, b), lambda i: (i, 0, 0)),
        out_shape=jax.ShapeDtypeStruct((l, EDIM, b), jnp.float32),
    )(x128, w)


def kernel(prem, hypo, table, W):
    B, L = prem.shape
    pairs = _tc_table_pack(table.T)
    table_flat = pairs.reshape(pairs.shape[0] * 4, HD)
    outs = []
    for ind in (prem, hypo):
        idx = ind.T.reshape(4 * L, B // 4)
        ridx = (idx // VB) * VB + 4 * (idx % QV) + (idx % VB) // QV
        emb = _sc_gather(ridx, table_flat)
        out_t = _tc_project_t(emb, W, L, B)
        outs.append(out_t.transpose(2, 0, 1))
    return (outs[0], outs[1])
